# baseline (speedup 1.0000x reference)
import functools

import jax
import jax.numpy as jnp
from jax import lax
from jax.experimental import pallas as pl
from jax.experimental.pallas import tpu as pltpu
from jax.experimental.pallas import tpu_sc as plsc

N_NODES = 10000
N_EDGES = 320000
D = 128
DA = 144
N_GRAPHS = 64
N_CLASSES = 40

NC = 2
NS = 16
NW = NC * NS

N_PAD = 10240
RPS = N_PAD // NS
CHUNK = 128
CPW0 = 128
CPW1 = 32
N_CHUNKS = NS * (CPW0 + CPW1)
E_PAD = N_CHUNKS * CHUNK

BN = 1280
GRID = N_PAD // BN


GSUB = 4
SUBC = CHUNK // GSUB


def _sc_agg_body(d, table, src_e, dst_e, zeros, out,
                 acc_sh, idx_s, idx_d, rows0, rows1, *sems):
    rows = (rows0, rows1)
    isem = sems[:4]
    gsem = (sems[4:4 + GSUB], sems[4 + GSUB:4 + 2 * GSUB])
    ssem = sems[4 + 2 * GSUB:]

    c = lax.axis_index("c")
    s = lax.axis_index("s")
    start = jnp.where(c == 0, s * CPW0, NS * CPW0 + s * CPW1)
    t4 = jnp.where(c == 0, CPW0 // 4, CPW1 // 4)

    pltpu.sync_copy(zeros.at[pl.ds(s * RPS, RPS)], acc_sh.at[pl.ds(s * RPS, RPS)])
    plsc.subcore_barrier()

    def fire_idx(i, b4):
        pltpu.async_copy(src_e.at[start + i], idx_s.at[b4], isem[b4])
        pltpu.async_copy(dst_e.at[start + i], idx_d.at[b4], isem[b4])

    def wait_idx(i, b4):
        pltpu.make_async_copy(src_e.at[start + i], idx_s.at[b4],
                              isem[b4]).wait()
        pltpu.make_async_copy(dst_e.at[start + i], idx_d.at[b4],
                              isem[b4]).wait()

    def fire_gather(i, b4, b2):
        for j in range(GSUB):
            pltpu.async_copy(table.at[idx_s.at[b4, 0, pl.ds(j * SUBC, SUBC)]],
                             rows[b2].at[pl.ds(j * SUBC, SUBC)], gsem[b2][j])

    def wait_gather(i, b4, b2):
        for j in range(GSUB):
            pltpu.make_async_copy(
                table.at[idx_s.at[b4, 0, pl.ds(j * SUBC, SUBC)]],
                rows[b2].at[pl.ds(j * SUBC, SUBC)], gsem[b2][j]).wait()

    def fire_scatter(i, b4, b2):
        pltpu.async_copy(rows[b2], acc_sh.at[idx_d.at[b4, 0]], ssem[b2],
                         add=True)

    def wait_scatter(i, b4, b2):
        pltpu.make_async_copy(rows[b2], acc_sh.at[idx_d.at[b4, 0]],
                              ssem[b2]).wait()

    @pl.when(t4 > 0)
    def _():
        fire_idx(0, 0)
        fire_idx(1, 1)
        wait_idx(0, 0)
        fire_gather(0, 0, 0)

    def loop_body(g, carry):
        for b in range(4):
            p = 4 * g + b
            b4 = b
            b2 = b % 2
            nb4 = (b + 1) % 4
            nb2 = 1 - b2

            if b == 0:
                @pl.when(g > 0)
                def _():
                    wait_scatter(p - 1, 3, nb2)
            else:
                wait_scatter(p - 1, b - 1, nb2)

            if b == 3:
                @pl.when(g < t4 - 1)
                def _():
                    wait_idx(p + 1, nb4)
                    fire_gather(p + 1, nb4, nb2)
                    fire_idx(p + 2, (b + 2) % 4)
            else:
                wait_idx(p + 1, nb4)
                fire_gather(p + 1, nb4, nb2)
                if b == 2:
                    @pl.when(g < t4 - 1)
                    def _():
                        fire_idx(p + 2, (b + 2) % 4)
                else:
                    fire_idx(p + 2, (b + 2) % 4)

            wait_gather(p, b4, b2)
            fire_scatter(p, b4, b2)
        return carry

    lax.fori_loop(0, t4, loop_body, 0)

    @pl.when(t4 > 0)
    def _():
        wait_scatter(0, 3, 1)

    plsc.subcore_barrier()
    pltpu.sync_copy(acc_sh.at[pl.ds(s * RPS, RPS)],
                    out.at[c, pl.ds(s * RPS, RPS)])


def _make_sc_agg(d):
    mesh = plsc.VectorSubcoreMesh(core_axis_name="c", subcore_axis_name="s",
                                  num_cores=NC, num_subcores=NS)
    out_type = jax.ShapeDtypeStruct((NC, N_PAD, d), jnp.float32)
    scratch = (
        [pltpu.VMEM_SHARED((N_PAD, d), jnp.float32),
         pltpu.VMEM((4, 1, CHUNK), jnp.int32),
         pltpu.VMEM((4, 1, CHUNK), jnp.int32)]
        + [pltpu.VMEM((CHUNK, d), jnp.float32)] * 2
        + [pltpu.SemaphoreType.DMA] * (6 + 2 * GSUB)
    )
    return pl.kernel(functools.partial(_sc_agg_body, d),
                     out_type=out_type, mesh=mesh, scratch_types=scratch,
                     compiler_params=pltpu.CompilerParams(
                         use_tc_tiling_on_sc=False))


def _tc_layer_body(acc_ref, xin_ref, wl_ref, wr_ref, bl_ref, out_ref):
    cnt = acc_ref[0, :, D] + acc_ref[1, :, D]
    agg = (acc_ref[0, :, :D] + acc_ref[1, :, :D]) / jnp.clip(cnt, 1.0, None)[:, None]
    h = (jnp.dot(agg, wl_ref[...], preferred_element_type=jnp.float32)
         + bl_ref[...]
         + jnp.dot(xin_ref[...], wr_ref[...], preferred_element_type=jnp.float32))
    out_ref[...] = jnp.maximum(h, 0.0)


def _make_tc_layer():
    return pl.pallas_call(
        _tc_layer_body,
        grid=(GRID,),
        in_specs=[
            pl.BlockSpec((NC, BN, DA), lambda i: (0, i, 0)),
            pl.BlockSpec((BN, D), lambda i: (i, 0)),
            pl.BlockSpec((D, D), lambda i: (0, 0)),
            pl.BlockSpec((D, D), lambda i: (0, 0)),
            pl.BlockSpec((1, D), lambda i: (0, 0)),
        ],
        out_specs=pl.BlockSpec((BN, D), lambda i: (i, 0)),
        out_shape=jax.ShapeDtypeStruct((N_PAD, D), jnp.float32),
    )


def _tc_final_body(acc_ref, cnt_ref, h_ref, wl_ref, wr_ref, bl_ref,
                   batch_ref, wlin_ref, blin_ref, out_ref, pool_acc, gcnt_acc):
    i = pl.program_id(0)

    @pl.when(i == 0)
    def _():
        pool_acc[...] = jnp.zeros_like(pool_acc)
        gcnt_acc[...] = jnp.zeros_like(gcnt_acc)

    cnt = cnt_ref[0, 0, 0, :] + cnt_ref[1, 0, 0, :]
    agg = (acc_ref[0] + acc_ref[1]) / jnp.clip(cnt, 1.0, None)[:, None]
    h2 = (jnp.dot(agg, wl_ref[...], preferred_element_type=jnp.float32)
          + bl_ref[...]
          + jnp.dot(h_ref[...], wr_ref[...], preferred_element_type=jnp.float32))
    b = batch_ref[0, 0, :]
    gids = lax.broadcasted_iota(jnp.int32, (N_GRAPHS, BN), 0)
    m = (gids == b[None, :]).astype(jnp.float32)
    pool_acc[...] += jnp.dot(m, h2, preferred_element_type=jnp.float32)
    gcnt_acc[...] += jnp.broadcast_to(jnp.sum(m, axis=1)[:, None], (N_GRAPHS, D))

    @pl.when(i == pl.num_programs(0) - 1)
    def _():
        pooled = pool_acc[...] / jnp.clip(gcnt_acc[...], 1.0, None)
        out_ref[...] = (jnp.dot(pooled, wlin_ref[...],
                                preferred_element_type=jnp.float32) + blin_ref[...])


def _make_tc_final():
    return pl.pallas_call(
        _tc_final_body,
        grid=(GRID,),
        in_specs=[
            pl.BlockSpec((NC, BN, D), lambda i: (0, i, 0)),
            pl.BlockSpec((NC, 1, 1, BN), lambda i: (0, i, 0, 0)),
            pl.BlockSpec((BN, D), lambda i: (i, 0)),
            pl.BlockSpec((D, D), lambda i: (0, 0)),
            pl.BlockSpec((D, D), lambda i: (0, 0)),
            pl.BlockSpec((1, D), lambda i: (0, 0)),
            pl.BlockSpec((1, 1, BN), lambda i: (i, 0, 0)),
            pl.BlockSpec((D, D), lambda i: (0, 0)),
            pl.BlockSpec((1, D), lambda i: (0, 0)),
        ],
        out_specs=pl.BlockSpec((N_GRAPHS, D), lambda i: (0, 0)),
        out_shape=jax.ShapeDtypeStruct((N_GRAPHS, D), jnp.float32),
        scratch_shapes=[
            pltpu.VMEM((N_GRAPHS, D), jnp.float32),
            pltpu.VMEM((N_GRAPHS, D), jnp.float32),
        ],
    )


_sc_agg_a = _make_sc_agg(DA)
_sc_agg_b = _make_sc_agg(D)
_tc_layer1 = _make_tc_layer()
_tc_final = _make_tc_final()


def kernel(x, edge_index, batch, Wl1, bl1, Wr1, Wl2, bl2, Wr2, Wlin, blin):
    x = x.astype(jnp.float32)
    src = edge_index[0].astype(jnp.int32)
    dst = edge_index[1].astype(jnp.int32)
    src_p = jnp.concatenate(
        [src, jnp.zeros((E_PAD - N_EDGES,),
                        jnp.int32)]).reshape(N_CHUNKS, 1, CHUNK)
    pad_dst = N_NODES + jnp.arange(E_PAD - N_EDGES, dtype=jnp.int32) % (
        N_PAD - N_NODES)
    dst_p = jnp.concatenate([dst, pad_dst]).reshape(N_CHUNKS, 1, CHUNK)
    x_p = jnp.concatenate(
        [x, jnp.zeros((N_PAD - N_NODES, D), jnp.float32)], axis=0)
    x_aug = jnp.concatenate(
        [x_p, jnp.ones((N_PAD, 1), jnp.float32),
         jnp.zeros((N_PAD, DA - D - 1), jnp.float32)], axis=1)
    zeros_a = jnp.zeros((N_PAD, DA), jnp.float32)
    zeros_b = jnp.zeros((N_PAD, D), jnp.float32)
    batch_p = jnp.concatenate(
        [batch.astype(jnp.int32),
         jnp.full((N_PAD - N_NODES,), N_GRAPHS, jnp.int32)]).reshape(GRID, 1, BN)

    wl1t = Wl1.T.astype(jnp.float32)
    wr1t = Wr1.T.astype(jnp.float32)
    wl2t = Wl2.T.astype(jnp.float32)
    wr2t = Wr2.T.astype(jnp.float32)
    bl1r = bl1.astype(jnp.float32).reshape(1, D)
    bl2r = bl2.astype(jnp.float32).reshape(1, D)
    wlint = jnp.pad(Wlin.T.astype(jnp.float32), ((0, 0), (0, D - N_CLASSES)))
    blinr = jnp.pad(blin.astype(jnp.float32), (0, D - N_CLASSES)).reshape(1, D)

    acc1 = _sc_agg_a(x_aug, src_p, dst_p, zeros_a)
    h = _tc_layer1(acc1, x_p, wl1t, wr1t, bl1r)
    acc2 = _sc_agg_b(h, src_p, dst_p, zeros_b)
    cntc = acc1[:, :, D].reshape(NC, GRID, 1, BN)
    out = _tc_final(acc2, cntc, h, wl2t, wr2t, bl2r, batch_p, wlint, blinr)
    return out[:, :N_CLASSES]


# speedup vs baseline: 1.3988x; 1.3988x over previous
import functools

import jax
import jax.numpy as jnp
from jax import lax
from jax.experimental import pallas as pl
from jax.experimental.pallas import tpu as pltpu
from jax.experimental.pallas import tpu_sc as plsc

N_NODES = 10000
N_EDGES = 320000
D = 128
DA = 160
N_GRAPHS = 64
N_CLASSES = 40

NC = 2
NS = 16
NW = NC * NS

N_PAD = 10240
RPS = N_PAD // NS
CHUNK = 128
CPW0 = 128
CPW1 = 32
N_CHUNKS = NS * (CPW0 + CPW1)
E_PAD = N_CHUNKS * CHUNK

BN = 1280
GRID = N_PAD // BN


GSUB = 4
SUBC = CHUNK // GSUB


def _sc_agg_body(d, table, src_e, dst_e, zeros, out,
                 acc0_sh, acc1_sh, idx_s, idx_d, rows0, rows1, *sems):
    accs = (acc0_sh, acc1_sh)
    rows = (rows0, rows1)
    isem = sems[:4]
    gsem = (sems[4:4 + GSUB], sems[4 + GSUB:4 + 2 * GSUB])
    ssem = sems[4 + 2 * GSUB:]

    c = lax.axis_index("c")
    s = lax.axis_index("s")
    start = jnp.where(c == 0, s * CPW0, NS * CPW0 + s * CPW1)
    t4 = jnp.where(c == 0, CPW0 // 4, CPW1 // 4)

    pltpu.sync_copy(zeros.at[pl.ds(s * RPS, RPS)],
                    acc0_sh.at[pl.ds(s * RPS, RPS)])
    pltpu.sync_copy(zeros.at[pl.ds(s * RPS, RPS)],
                    acc1_sh.at[pl.ds(s * RPS, RPS)])
    plsc.subcore_barrier()

    def fire_idx(i, b4):
        pltpu.async_copy(src_e.at[start + i], idx_s.at[b4], isem[b4])
        pltpu.async_copy(dst_e.at[start + i], idx_d.at[b4], isem[b4])

    def wait_idx(i, b4):
        pltpu.make_async_copy(src_e.at[start + i], idx_s.at[b4],
                              isem[b4]).wait()
        pltpu.make_async_copy(dst_e.at[start + i], idx_d.at[b4],
                              isem[b4]).wait()

    def fire_gather(i, b4, b2):
        for j in range(GSUB):
            pltpu.async_copy(table.at[idx_s.at[b4, 0, pl.ds(j * SUBC, SUBC)]],
                             rows[b2].at[pl.ds(j * SUBC, SUBC)], gsem[b2][j])

    def wait_gather(i, b4, b2):
        for j in range(GSUB):
            pltpu.make_async_copy(
                table.at[idx_s.at[b4, 0, pl.ds(j * SUBC, SUBC)]],
                rows[b2].at[pl.ds(j * SUBC, SUBC)], gsem[b2][j]).wait()

    def fire_scatter(i, b4, b2):
        pltpu.async_copy(rows[b2], accs[b2].at[idx_d.at[b4, 0]], ssem[b2],
                         add=True)

    def wait_scatter(i, b4, b2):
        pltpu.make_async_copy(rows[b2], accs[b2].at[idx_d.at[b4, 0]],
                              ssem[b2]).wait()

    @pl.when(t4 > 0)
    def _():
        fire_idx(0, 0)
        fire_idx(1, 1)
        wait_idx(0, 0)
        fire_gather(0, 0, 0)

    def loop_body(g, carry):
        for b in range(4):
            p = 4 * g + b
            b4 = b
            b2 = b % 2
            nb4 = (b + 1) % 4
            nb2 = 1 - b2

            if b == 0:
                @pl.when(g > 0)
                def _():
                    wait_scatter(p - 1, 3, nb2)
            else:
                wait_scatter(p - 1, b - 1, nb2)

            if b == 3:
                @pl.when(g < t4 - 1)
                def _():
                    wait_idx(p + 1, nb4)
                    fire_gather(p + 1, nb4, nb2)
                    fire_idx(p + 2, (b + 2) % 4)
            else:
                wait_idx(p + 1, nb4)
                fire_gather(p + 1, nb4, nb2)
                if b == 2:
                    @pl.when(g < t4 - 1)
                    def _():
                        fire_idx(p + 2, (b + 2) % 4)
                else:
                    fire_idx(p + 2, (b + 2) % 4)

            wait_gather(p, b4, b2)
            fire_scatter(p, b4, b2)
        return carry

    lax.fori_loop(0, t4, loop_body, 0)

    @pl.when(t4 > 0)
    def _():
        wait_scatter(0, 3, 1)

    plsc.subcore_barrier()
    pltpu.sync_copy(acc0_sh.at[pl.ds(s * RPS, RPS)],
                    out.at[c, 0, pl.ds(s * RPS, RPS)])
    pltpu.sync_copy(acc1_sh.at[pl.ds(s * RPS, RPS)],
                    out.at[c, 1, pl.ds(s * RPS, RPS)])


def _make_sc_agg(d):
    mesh = plsc.VectorSubcoreMesh(core_axis_name="c", subcore_axis_name="s",
                                  num_cores=NC, num_subcores=NS)
    out_type = jax.ShapeDtypeStruct((NC, 2, N_PAD, d), jnp.bfloat16)
    scratch = (
        [pltpu.VMEM_SHARED((N_PAD, d), jnp.bfloat16),
         pltpu.VMEM_SHARED((N_PAD, d), jnp.bfloat16),
         pltpu.VMEM((4, 1, CHUNK), jnp.int32),
         pltpu.VMEM((4, 1, CHUNK), jnp.int32)]
        + [pltpu.VMEM((CHUNK, d), jnp.bfloat16)] * 2
        + [pltpu.SemaphoreType.DMA] * (6 + 2 * GSUB)
    )
    return pl.kernel(functools.partial(_sc_agg_body, d),
                     out_type=out_type, mesh=mesh, scratch_types=scratch,
                     compiler_params=pltpu.CompilerParams(
                         use_tc_tiling_on_sc=False))


def _tc_layer_body(acc_ref, xin_ref, wl_ref, wr_ref, bl_ref, out_ref, outb_ref):
    acc = acc_ref[...].astype(jnp.float32)
    ssum = acc[0, 0] + acc[0, 1] + acc[1, 0] + acc[1, 1]
    cnt = ssum[:, D]
    agg = ssum[:, :D] / jnp.clip(cnt, 1.0, None)[:, None]
    h = (jnp.dot(agg, wl_ref[...], preferred_element_type=jnp.float32)
         + bl_ref[...]
         + jnp.dot(xin_ref[...], wr_ref[...], preferred_element_type=jnp.float32))
    h = jnp.maximum(h, 0.0)
    out_ref[...] = h
    outb_ref[...] = h.astype(jnp.bfloat16)


def _make_tc_layer():
    return pl.pallas_call(
        _tc_layer_body,
        grid=(GRID,),
        in_specs=[
            pl.BlockSpec((NC, 2, BN, DA), lambda i: (0, 0, i, 0)),
            pl.BlockSpec((BN, D), lambda i: (i, 0)),
            pl.BlockSpec((D, D), lambda i: (0, 0)),
            pl.BlockSpec((D, D), lambda i: (0, 0)),
            pl.BlockSpec((1, D), lambda i: (0, 0)),
        ],
        out_specs=[pl.BlockSpec((BN, D), lambda i: (i, 0)),
                   pl.BlockSpec((BN, D), lambda i: (i, 0))],
        out_shape=[jax.ShapeDtypeStruct((N_PAD, D), jnp.float32),
                   jax.ShapeDtypeStruct((N_PAD, D), jnp.bfloat16)],
    )


def _tc_final_body(acc_ref, cnt_ref, h_ref, wl_ref, wr_ref, bl_ref,
                   batch_ref, wlin_ref, blin_ref, out_ref, pool_acc, gcnt_acc):
    i = pl.program_id(0)

    @pl.when(i == 0)
    def _():
        pool_acc[...] = jnp.zeros_like(pool_acc)
        gcnt_acc[...] = jnp.zeros_like(gcnt_acc)

    cnt = (cnt_ref[0, 0, 0, :] + cnt_ref[1, 0, 0, :]
           + cnt_ref[2, 0, 0, :] + cnt_ref[3, 0, 0, :]).astype(jnp.float32)
    acc = acc_ref[...].astype(jnp.float32)
    agg = (acc[0, 0] + acc[0, 1] + acc[1, 0]
           + acc[1, 1]) / jnp.clip(cnt, 1.0, None)[:, None]
    h2 = (jnp.dot(agg, wl_ref[...], preferred_element_type=jnp.float32)
          + bl_ref[...]
          + jnp.dot(h_ref[...], wr_ref[...], preferred_element_type=jnp.float32))
    b = batch_ref[0, 0, :]
    gids = lax.broadcasted_iota(jnp.int32, (N_GRAPHS, BN), 0)
    m = (gids == b[None, :]).astype(jnp.float32)
    pool_acc[...] += jnp.dot(m, h2, preferred_element_type=jnp.float32)
    gcnt_acc[...] += jnp.broadcast_to(jnp.sum(m, axis=1)[:, None], (N_GRAPHS, D))

    @pl.when(i == pl.num_programs(0) - 1)
    def _():
        pooled = pool_acc[...] / jnp.clip(gcnt_acc[...], 1.0, None)
        out_ref[...] = (jnp.dot(pooled, wlin_ref[...],
                                preferred_element_type=jnp.float32) + blin_ref[...])


def _make_tc_final():
    return pl.pallas_call(
        _tc_final_body,
        grid=(GRID,),
        in_specs=[
            pl.BlockSpec((NC, 2, BN, D), lambda i: (0, 0, i, 0)),
            pl.BlockSpec((NC * 2, 1, 1, BN), lambda i: (0, i, 0, 0)),
            pl.BlockSpec((BN, D), lambda i: (i, 0)),
            pl.BlockSpec((D, D), lambda i: (0, 0)),
            pl.BlockSpec((D, D), lambda i: (0, 0)),
            pl.BlockSpec((1, D), lambda i: (0, 0)),
            pl.BlockSpec((1, 1, BN), lambda i: (i, 0, 0)),
            pl.BlockSpec((D, D), lambda i: (0, 0)),
            pl.BlockSpec((1, D), lambda i: (0, 0)),
        ],
        out_specs=pl.BlockSpec((N_GRAPHS, D), lambda i: (0, 0)),
        out_shape=jax.ShapeDtypeStruct((N_GRAPHS, D), jnp.float32),
        scratch_shapes=[
            pltpu.VMEM((N_GRAPHS, D), jnp.float32),
            pltpu.VMEM((N_GRAPHS, D), jnp.float32),
        ],
    )


_sc_agg_a = _make_sc_agg(DA)
_sc_agg_b = _make_sc_agg(D)
_tc_layer1 = _make_tc_layer()
_tc_final = _make_tc_final()


def kernel(x, edge_index, batch, Wl1, bl1, Wr1, Wl2, bl2, Wr2, Wlin, blin):
    x = x.astype(jnp.float32)
    src = edge_index[0].astype(jnp.int32)
    dst = edge_index[1].astype(jnp.int32)
    src_p = jnp.concatenate(
        [src, jnp.zeros((E_PAD - N_EDGES,),
                        jnp.int32)]).reshape(N_CHUNKS, 1, CHUNK)
    pad_dst = N_NODES + jnp.arange(E_PAD - N_EDGES, dtype=jnp.int32) % (
        N_PAD - N_NODES)
    dst_p = jnp.concatenate([dst, pad_dst]).reshape(N_CHUNKS, 1, CHUNK)
    x_p = jnp.concatenate(
        [x, jnp.zeros((N_PAD - N_NODES, D), jnp.float32)], axis=0)
    x_aug = jnp.concatenate(
        [x_p.astype(jnp.bfloat16),
         jnp.ones((N_PAD, 1), jnp.bfloat16),
         jnp.zeros((N_PAD, DA - D - 1), jnp.bfloat16)], axis=1)
    zeros_a = jnp.zeros((N_PAD, DA), jnp.bfloat16)
    zeros_b = jnp.zeros((N_PAD, D), jnp.bfloat16)
    batch_p = jnp.concatenate(
        [batch.astype(jnp.int32),
         jnp.full((N_PAD - N_NODES,), N_GRAPHS, jnp.int32)]).reshape(GRID, 1, BN)

    wl1t = Wl1.T.astype(jnp.float32)
    wr1t = Wr1.T.astype(jnp.float32)
    wl2t = Wl2.T.astype(jnp.float32)
    wr2t = Wr2.T.astype(jnp.float32)
    bl1r = bl1.astype(jnp.float32).reshape(1, D)
    bl2r = bl2.astype(jnp.float32).reshape(1, D)
    wlint = jnp.pad(Wlin.T.astype(jnp.float32), ((0, 0), (0, D - N_CLASSES)))
    blinr = jnp.pad(blin.astype(jnp.float32), (0, D - N_CLASSES)).reshape(1, D)

    acc1 = _sc_agg_a(x_aug, src_p, dst_p, zeros_a)
    h, hbf = _tc_layer1(acc1, x_p, wl1t, wr1t, bl1r)
    acc2 = _sc_agg_b(hbf, src_p, dst_p, zeros_b)
    cntc = acc1[:, :, :, D].reshape(NC * 2, GRID, 1, BN)
    out = _tc_final(acc2, cntc, h, wl2t, wr2t, bl2r, batch_p, wlint, blinr)
    return out[:, :N_CLASSES]


# speedup vs baseline: 1.7126x; 1.2244x over previous
import functools

import jax
import jax.numpy as jnp
from jax import lax
from jax.experimental import pallas as pl
from jax.experimental.pallas import tpu as pltpu
from jax.experimental.pallas import tpu_sc as plsc

N_NODES = 10000
N_EDGES = 320000
D = 128
DA = 160
N_GRAPHS = 64
N_CLASSES = 40

NC = 2
NS = 16
NW = NC * NS

N_PAD = 10240
RPS = N_PAD // NS
CHUNK = 128
CPW0 = 128
CPW1 = 32
N_CHUNKS = NS * (CPW0 + CPW1)
E_PAD = N_CHUNKS * CHUNK

BN = 1280
GRID = N_PAD // BN


GSUB = 4
SUBC = CHUNK // GSUB


def _sc_agg_body(d, with_cnt, *refs):
    if with_cnt:
        (table, src_e, dst_e, zeros, zeros16, ones16, out, cnt_out,
         acc0_sh, acc1_sh, cnt_sh, idx_s, idx_d, rows0, rows1, ones_v,
         *sems) = refs
    else:
        (table, src_e, dst_e, zeros, out,
         acc0_sh, acc1_sh, idx_s, idx_d, rows0, rows1, *sems) = refs
    accs = (acc0_sh, acc1_sh)
    rows = (rows0, rows1)
    isem = sems[:4]
    gsem = (sems[4:4 + GSUB], sems[4 + GSUB:4 + 2 * GSUB])
    ssem = sems[4 + 2 * GSUB:4 + 2 * GSUB + 2]
    csem = sems[4 + 2 * GSUB + 2:]

    c = lax.axis_index("c")
    s = lax.axis_index("s")
    start = jnp.where(c == 0, s * CPW0, NS * CPW0 + s * CPW1)
    t4 = jnp.where(c == 0, CPW0 // 4, CPW1 // 4)

    pltpu.sync_copy(zeros.at[pl.ds(s * RPS, RPS)],
                    acc0_sh.at[pl.ds(s * RPS, RPS)])
    pltpu.sync_copy(zeros.at[pl.ds(s * RPS, RPS)],
                    acc1_sh.at[pl.ds(s * RPS, RPS)])
    if with_cnt:
        pltpu.sync_copy(zeros16.at[pl.ds(s * RPS, RPS)],
                        cnt_sh.at[pl.ds(s * RPS, RPS)])
        pltpu.sync_copy(ones16, ones_v)
    plsc.subcore_barrier()

    def fire_idx(i, b4):
        pltpu.async_copy(src_e.at[start + i], idx_s.at[b4], isem[b4])
        pltpu.async_copy(dst_e.at[start + i], idx_d.at[b4], isem[b4])

    def wait_idx(i, b4):
        pltpu.make_async_copy(src_e.at[start + i], idx_s.at[b4],
                              isem[b4]).wait()
        pltpu.make_async_copy(dst_e.at[start + i], idx_d.at[b4],
                              isem[b4]).wait()

    def fire_gather(i, b4, b2):
        for j in range(GSUB):
            pltpu.async_copy(table.at[idx_s.at[b4, 0, pl.ds(j * SUBC, SUBC)]],
                             rows[b2].at[pl.ds(j * SUBC, SUBC)], gsem[b2][j])

    def wait_gather(i, b4, b2):
        for j in range(GSUB):
            pltpu.make_async_copy(
                table.at[idx_s.at[b4, 0, pl.ds(j * SUBC, SUBC)]],
                rows[b2].at[pl.ds(j * SUBC, SUBC)], gsem[b2][j]).wait()

    def fire_scatter(i, b4, b2):
        pltpu.async_copy(rows[b2], accs[b2].at[idx_d.at[b4, 0]], ssem[b2],
                         add=True)
        if with_cnt:
            pltpu.async_copy(ones_v, cnt_sh.at[idx_d.at[b4, 0]], csem[b2],
                             add=True)

    def wait_scatter(i, b4, b2):
        pltpu.make_async_copy(rows[b2], accs[b2].at[idx_d.at[b4, 0]],
                              ssem[b2]).wait()
        if with_cnt:
            pltpu.make_async_copy(ones_v, cnt_sh.at[idx_d.at[b4, 0]],
                                  csem[b2]).wait()

    @pl.when(t4 > 0)
    def _():
        fire_idx(0, 0)
        fire_idx(1, 1)
        wait_idx(0, 0)
        fire_gather(0, 0, 0)

    def loop_body(g, carry):
        for b in range(4):
            p = 4 * g + b
            b4 = b
            b2 = b % 2
            nb4 = (b + 1) % 4
            nb2 = 1 - b2

            if b == 0:
                @pl.when(g > 0)
                def _():
                    wait_scatter(p - 1, 3, nb2)
            else:
                wait_scatter(p - 1, b - 1, nb2)

            if b == 3:
                @pl.when(g < t4 - 1)
                def _():
                    wait_idx(p + 1, nb4)
                    fire_gather(p + 1, nb4, nb2)
                    fire_idx(p + 2, (b + 2) % 4)
            else:
                wait_idx(p + 1, nb4)
                fire_gather(p + 1, nb4, nb2)
                if b == 2:
                    @pl.when(g < t4 - 1)
                    def _():
                        fire_idx(p + 2, (b + 2) % 4)
                else:
                    fire_idx(p + 2, (b + 2) % 4)

            wait_gather(p, b4, b2)
            fire_scatter(p, b4, b2)
        return carry

    lax.fori_loop(0, t4, loop_body, 0)

    @pl.when(t4 > 0)
    def _():
        wait_scatter(0, 3, 1)

    plsc.subcore_barrier()
    pltpu.sync_copy(acc0_sh.at[pl.ds(s * RPS, RPS)],
                    out.at[c, 0, pl.ds(s * RPS, RPS)])
    pltpu.sync_copy(acc1_sh.at[pl.ds(s * RPS, RPS)],
                    out.at[c, 1, pl.ds(s * RPS, RPS)])
    if with_cnt:
        pltpu.sync_copy(cnt_sh.at[pl.ds(s * RPS, RPS)],
                        cnt_out.at[c, pl.ds(s * RPS, RPS)])


def _make_sc_agg(d, with_cnt):
    mesh = plsc.VectorSubcoreMesh(core_axis_name="c", subcore_axis_name="s",
                                  num_cores=NC, num_subcores=NS)
    out_type = [jax.ShapeDtypeStruct((NC, 2, N_PAD, d), jnp.bfloat16)]
    if with_cnt:
        out_type.append(jax.ShapeDtypeStruct((NC, N_PAD, 16), jnp.bfloat16))
    scratch = [pltpu.VMEM_SHARED((N_PAD, d), jnp.bfloat16),
               pltpu.VMEM_SHARED((N_PAD, d), jnp.bfloat16)]
    if with_cnt:
        scratch.append(pltpu.VMEM_SHARED((N_PAD, 16), jnp.bfloat16))
    scratch += [pltpu.VMEM((4, 1, CHUNK), jnp.int32),
                pltpu.VMEM((4, 1, CHUNK), jnp.int32)]
    scratch += [pltpu.VMEM((CHUNK, d), jnp.bfloat16)] * 2
    if with_cnt:
        scratch.append(pltpu.VMEM((CHUNK, 16), jnp.bfloat16))
    scratch += [pltpu.SemaphoreType.DMA] * (6 + 2 * GSUB
                                            + (2 if with_cnt else 0))
    return pl.kernel(functools.partial(_sc_agg_body, d, with_cnt),
                     out_type=out_type, mesh=mesh, scratch_types=scratch,
                     compiler_params=pltpu.CompilerParams(
                         use_tc_tiling_on_sc=False))


def _tc_layer_body(acc_ref, cnt_ref, xin_ref, wl_ref, wr_ref, bl_ref,
                   out_ref, outb_ref):
    acc = acc_ref[...].astype(jnp.float32)
    ssum = acc[0, 0] + acc[0, 1] + acc[1, 0] + acc[1, 1]
    cnt = (cnt_ref[0, :, 0] + cnt_ref[1, :, 0]).astype(jnp.float32)
    agg = ssum / jnp.clip(cnt, 1.0, None)[:, None]
    h = (jnp.dot(agg, wl_ref[...], preferred_element_type=jnp.float32)
         + bl_ref[...]
         + jnp.dot(xin_ref[...], wr_ref[...], preferred_element_type=jnp.float32))
    h = jnp.maximum(h, 0.0)
    out_ref[...] = h
    outb_ref[...] = h.astype(jnp.bfloat16)


def _make_tc_layer():
    return pl.pallas_call(
        _tc_layer_body,
        grid=(GRID,),
        in_specs=[
            pl.BlockSpec((NC, 2, BN, D), lambda i: (0, 0, i, 0)),
            pl.BlockSpec((NC, BN, 16), lambda i: (0, i, 0)),
            pl.BlockSpec((BN, D), lambda i: (i, 0)),
            pl.BlockSpec((D, D), lambda i: (0, 0)),
            pl.BlockSpec((D, D), lambda i: (0, 0)),
            pl.BlockSpec((1, D), lambda i: (0, 0)),
        ],
        out_specs=[pl.BlockSpec((BN, D), lambda i: (i, 0)),
                   pl.BlockSpec((BN, D), lambda i: (i, 0))],
        out_shape=[jax.ShapeDtypeStruct((N_PAD, D), jnp.float32),
                   jax.ShapeDtypeStruct((N_PAD, D), jnp.bfloat16)],
    )


def _tc_final_body(acc_ref, cnt_ref, h_ref, wl_ref, wr_ref, bl_ref,
                   batch_ref, wlin_ref, blin_ref, out_ref, pool_acc, gcnt_acc):
    i = pl.program_id(0)

    @pl.when(i == 0)
    def _():
        pool_acc[...] = jnp.zeros_like(pool_acc)
        gcnt_acc[...] = jnp.zeros_like(gcnt_acc)

    cnt = (cnt_ref[0, :, 0] + cnt_ref[1, :, 0]).astype(jnp.float32)
    acc = acc_ref[...].astype(jnp.float32)
    agg = (acc[0, 0] + acc[0, 1] + acc[1, 0]
           + acc[1, 1]) / jnp.clip(cnt, 1.0, None)[:, None]
    h2 = (jnp.dot(agg, wl_ref[...], preferred_element_type=jnp.float32)
          + bl_ref[...]
          + jnp.dot(h_ref[...], wr_ref[...], preferred_element_type=jnp.float32))
    b = batch_ref[0, 0, :]
    gids = lax.broadcasted_iota(jnp.int32, (N_GRAPHS, BN), 0)
    m = (gids == b[None, :]).astype(jnp.float32)
    pool_acc[...] += jnp.dot(m, h2, preferred_element_type=jnp.float32)
    gcnt_acc[...] += jnp.broadcast_to(jnp.sum(m, axis=1)[:, None], (N_GRAPHS, D))

    @pl.when(i == pl.num_programs(0) - 1)
    def _():
        pooled = pool_acc[...] / jnp.clip(gcnt_acc[...], 1.0, None)
        out_ref[...] = (jnp.dot(pooled, wlin_ref[...],
                                preferred_element_type=jnp.float32) + blin_ref[...])


def _make_tc_final():
    return pl.pallas_call(
        _tc_final_body,
        grid=(GRID,),
        in_specs=[
            pl.BlockSpec((NC, 2, BN, D), lambda i: (0, 0, i, 0)),
            pl.BlockSpec((NC, BN, 16), lambda i: (0, i, 0)),
            pl.BlockSpec((BN, D), lambda i: (i, 0)),
            pl.BlockSpec((D, D), lambda i: (0, 0)),
            pl.BlockSpec((D, D), lambda i: (0, 0)),
            pl.BlockSpec((1, D), lambda i: (0, 0)),
            pl.BlockSpec((1, 1, BN), lambda i: (i, 0, 0)),
            pl.BlockSpec((D, D), lambda i: (0, 0)),
            pl.BlockSpec((1, D), lambda i: (0, 0)),
        ],
        out_specs=pl.BlockSpec((N_GRAPHS, D), lambda i: (0, 0)),
        out_shape=jax.ShapeDtypeStruct((N_GRAPHS, D), jnp.float32),
        scratch_shapes=[
            pltpu.VMEM((N_GRAPHS, D), jnp.float32),
            pltpu.VMEM((N_GRAPHS, D), jnp.float32),
        ],
    )


_sc_agg_a = _make_sc_agg(D, True)
_sc_agg_b = _make_sc_agg(D, False)
_tc_layer1 = _make_tc_layer()
_tc_final = _make_tc_final()


def kernel(x, edge_index, batch, Wl1, bl1, Wr1, Wl2, bl2, Wr2, Wlin, blin):
    x = x.astype(jnp.float32)
    src = edge_index[0].astype(jnp.int32)
    dst = edge_index[1].astype(jnp.int32)
    src_p = jnp.concatenate(
        [src, jnp.zeros((E_PAD - N_EDGES,),
                        jnp.int32)]).reshape(N_CHUNKS, 1, CHUNK)
    pad_dst = N_NODES + jnp.arange(E_PAD - N_EDGES, dtype=jnp.int32) % (
        N_PAD - N_NODES)
    dst_p = jnp.concatenate([dst, pad_dst]).reshape(N_CHUNKS, 1, CHUNK)
    x_p = jnp.concatenate(
        [x, jnp.zeros((N_PAD - N_NODES, D), jnp.float32)], axis=0)
    x_bf = x_p.astype(jnp.bfloat16)
    zeros_b = jnp.zeros((N_PAD, D), jnp.bfloat16)
    zeros16 = jnp.zeros((N_PAD, 16), jnp.bfloat16)
    ones16 = jnp.ones((CHUNK, 16), jnp.bfloat16)
    batch_p = jnp.concatenate(
        [batch.astype(jnp.int32),
         jnp.full((N_PAD - N_NODES,), N_GRAPHS, jnp.int32)]).reshape(GRID, 1, BN)

    wl1t = Wl1.T.astype(jnp.float32)
    wr1t = Wr1.T.astype(jnp.float32)
    wl2t = Wl2.T.astype(jnp.float32)
    wr2t = Wr2.T.astype(jnp.float32)
    bl1r = bl1.astype(jnp.float32).reshape(1, D)
    bl2r = bl2.astype(jnp.float32).reshape(1, D)
    wlint = jnp.pad(Wlin.T.astype(jnp.float32), ((0, 0), (0, D - N_CLASSES)))
    blinr = jnp.pad(blin.astype(jnp.float32), (0, D - N_CLASSES)).reshape(1, D)

    acc1, cnt16 = _sc_agg_a(x_bf, src_p, dst_p, zeros_b, zeros16, ones16)
    h, hbf = _tc_layer1(acc1, cnt16, x_p, wl1t, wr1t, bl1r)
    (acc2,) = _sc_agg_b(hbf, src_p, dst_p, zeros_b)
    out = _tc_final(acc2, cnt16, h, wl2t, wr2t, bl2r, batch_p, wlint, blinr)
    return out[:, :N_CLASSES]


# speedup vs baseline: 1.7129x; 1.0002x over previous
import functools

import jax
import jax.numpy as jnp
from jax import lax
from jax.experimental import pallas as pl
from jax.experimental.pallas import tpu as pltpu
from jax.experimental.pallas import tpu_sc as plsc

N_NODES = 10000
N_EDGES = 320000
D = 128
N_GRAPHS = 64
N_CLASSES = 40

NC = 2
NS = 16
NW = NC * NS

N_PAD = 10240
RPS = N_PAD // NS
CHUNK = 128
CPW0 = 128
CPW1 = 32
N_CHUNKS = NS * (CPW0 + CPW1)
E_PAD = N_CHUNKS * CHUNK

BN = 1280
GRID = N_PAD // BN


GSUB = 4
SUBC = CHUNK // GSUB


def _sc_agg_body(d, with_cnt, *refs):
    if with_cnt:
        (table, src_e, dst_e, zeros, zeros16, ones16, out, cnt_out,
         acc0_sh, acc1_sh, cnt_sh, idx_s, idx_d, rows0, rows1, ones_v,
         *sems) = refs
    else:
        (table, src_e, dst_e, zeros, out,
         acc0_sh, acc1_sh, idx_s, idx_d, rows0, rows1, *sems) = refs
    accs = (acc0_sh, acc1_sh)
    rows = (rows0, rows1)
    isem = sems[:4]
    gsem = (sems[4:4 + GSUB], sems[4 + GSUB:4 + 2 * GSUB])
    ssem = sems[4 + 2 * GSUB:4 + 2 * GSUB + 2]
    csem = sems[4 + 2 * GSUB + 2:]

    c = lax.axis_index("c")
    s = lax.axis_index("s")
    start = jnp.where(c == 0, s * CPW0, NS * CPW0 + s * CPW1)
    t4 = jnp.where(c == 0, CPW0 // 4, CPW1 // 4)

    pltpu.sync_copy(zeros.at[pl.ds(s * RPS, RPS)],
                    acc0_sh.at[pl.ds(s * RPS, RPS)])
    pltpu.sync_copy(zeros.at[pl.ds(s * RPS, RPS)],
                    acc1_sh.at[pl.ds(s * RPS, RPS)])
    if with_cnt:
        pltpu.sync_copy(zeros16.at[pl.ds(s * RPS, RPS)],
                        cnt_sh.at[pl.ds(s * RPS, RPS)])
        pltpu.sync_copy(ones16, ones_v)
    plsc.subcore_barrier()

    def fire_idx(i, b4):
        pltpu.async_copy(src_e.at[start + i], idx_s.at[b4], isem[b4])
        pltpu.async_copy(dst_e.at[start + i], idx_d.at[b4], isem[b4])

    def wait_idx(i, b4):
        pltpu.make_async_copy(src_e.at[start + i], idx_s.at[b4],
                              isem[b4]).wait()
        pltpu.make_async_copy(dst_e.at[start + i], idx_d.at[b4],
                              isem[b4]).wait()

    def fire_gather(i, b4, b2):
        for j in range(GSUB):
            pltpu.async_copy(table.at[idx_s.at[b4, 0, pl.ds(j * SUBC, SUBC)]],
                             rows[b2].at[pl.ds(j * SUBC, SUBC)], gsem[b2][j])

    def wait_gather(i, b4, b2):
        for j in range(GSUB):
            pltpu.make_async_copy(
                table.at[idx_s.at[b4, 0, pl.ds(j * SUBC, SUBC)]],
                rows[b2].at[pl.ds(j * SUBC, SUBC)], gsem[b2][j]).wait()

    def fire_scatter(i, b4, b2):
        pltpu.async_copy(rows[b2], accs[b2].at[idx_d.at[b4, 0]], ssem[b2],
                         add=True)
        if with_cnt:
            pltpu.async_copy(ones_v, cnt_sh.at[idx_d.at[b4, 0]], csem[b2],
                             add=True)

    def wait_scatter(i, b4, b2):
        pltpu.make_async_copy(rows[b2], accs[b2].at[idx_d.at[b4, 0]],
                              ssem[b2]).wait()
        if with_cnt:
            pltpu.make_async_copy(ones_v, cnt_sh.at[idx_d.at[b4, 0]],
                                  csem[b2]).wait()

    @pl.when(t4 > 0)
    def _():
        fire_idx(0, 0)
        fire_idx(1, 1)
        wait_idx(0, 0)
        fire_gather(0, 0, 0)

    def loop_body(g, carry):
        for b in range(4):
            p = 4 * g + b
            b4 = b
            b2 = b % 2
            nb4 = (b + 1) % 4
            nb2 = 1 - b2

            if b == 0:
                @pl.when(g > 0)
                def _():
                    wait_scatter(p - 1, 3, nb2)
            else:
                wait_scatter(p - 1, b - 1, nb2)

            if b == 3:
                @pl.when(g < t4 - 1)
                def _():
                    wait_idx(p + 1, nb4)
                    fire_gather(p + 1, nb4, nb2)
                    fire_idx(p + 2, (b + 2) % 4)
            else:
                wait_idx(p + 1, nb4)
                fire_gather(p + 1, nb4, nb2)
                if b == 2:
                    @pl.when(g < t4 - 1)
                    def _():
                        fire_idx(p + 2, (b + 2) % 4)
                else:
                    fire_idx(p + 2, (b + 2) % 4)

            wait_gather(p, b4, b2)
            fire_scatter(p, b4, b2)
        return carry

    lax.fori_loop(0, t4, loop_body, 0)

    @pl.when(t4 > 0)
    def _():
        wait_scatter(0, 3, 1)

    plsc.subcore_barrier()
    pltpu.sync_copy(acc0_sh.at[pl.ds(s * RPS, RPS)],
                    out.at[c, 0, pl.ds(s * RPS, RPS)])
    pltpu.sync_copy(acc1_sh.at[pl.ds(s * RPS, RPS)],
                    out.at[c, 1, pl.ds(s * RPS, RPS)])
    if with_cnt:
        pltpu.sync_copy(cnt_sh.at[pl.ds(s * RPS, RPS)],
                        cnt_out.at[c, pl.ds(s * RPS, RPS)])


def _make_sc_agg(d, with_cnt):
    mesh = plsc.VectorSubcoreMesh(core_axis_name="c", subcore_axis_name="s",
                                  num_cores=NC, num_subcores=NS)
    out_type = [jax.ShapeDtypeStruct((NC, 2, N_PAD, d), jnp.bfloat16)]
    if with_cnt:
        out_type.append(jax.ShapeDtypeStruct((NC, N_PAD, 16), jnp.bfloat16))
    scratch = [pltpu.VMEM_SHARED((N_PAD, d), jnp.bfloat16),
               pltpu.VMEM_SHARED((N_PAD, d), jnp.bfloat16)]
    if with_cnt:
        scratch.append(pltpu.VMEM_SHARED((N_PAD, 16), jnp.bfloat16))
    scratch += [pltpu.VMEM((4, 1, CHUNK), jnp.int32),
                pltpu.VMEM((4, 1, CHUNK), jnp.int32)]
    scratch += [pltpu.VMEM((CHUNK, d), jnp.bfloat16)] * 2
    if with_cnt:
        scratch.append(pltpu.VMEM((CHUNK, 16), jnp.bfloat16))
    scratch += [pltpu.SemaphoreType.DMA] * (6 + 2 * GSUB
                                            + (2 if with_cnt else 0))
    return pl.kernel(functools.partial(_sc_agg_body, d, with_cnt),
                     out_type=out_type, mesh=mesh, scratch_types=scratch,
                     compiler_params=pltpu.CompilerParams(
                         use_tc_tiling_on_sc=False))


def _tc_layer_body(acc_ref, cnt_ref, xin_ref, wl_ref, wr_ref, bl_ref,
                   out_ref, outb_ref):
    acc = acc_ref[...].astype(jnp.float32)
    ssum = acc[0, 0] + acc[0, 1] + acc[1, 0] + acc[1, 1]
    cnt = (cnt_ref[0, :, 0] + cnt_ref[1, :, 0]).astype(jnp.float32)
    agg = ssum / jnp.clip(cnt, 1.0, None)[:, None]
    h = (jnp.dot(agg, wl_ref[...], preferred_element_type=jnp.float32)
         + bl_ref[...]
         + jnp.dot(xin_ref[...], wr_ref[...], preferred_element_type=jnp.float32))
    h = jnp.maximum(h, 0.0)
    out_ref[...] = h
    outb_ref[...] = h.astype(jnp.bfloat16)


def _make_tc_layer():
    return pl.pallas_call(
        _tc_layer_body,
        grid=(GRID,),
        in_specs=[
            pl.BlockSpec((NC, 2, BN, D), lambda i: (0, 0, i, 0)),
            pl.BlockSpec((NC, BN, 16), lambda i: (0, i, 0)),
            pl.BlockSpec((BN, D), lambda i: (i, 0)),
            pl.BlockSpec((D, D), lambda i: (0, 0)),
            pl.BlockSpec((D, D), lambda i: (0, 0)),
            pl.BlockSpec((1, D), lambda i: (0, 0)),
        ],
        out_specs=[pl.BlockSpec((BN, D), lambda i: (i, 0)),
                   pl.BlockSpec((BN, D), lambda i: (i, 0))],
        out_shape=[jax.ShapeDtypeStruct((N_PAD, D), jnp.float32),
                   jax.ShapeDtypeStruct((N_PAD, D), jnp.bfloat16)],
    )


def _tc_final_body(acc_ref, cnt_ref, h_ref, wl_ref, wr_ref, bl_ref,
                   batch_ref, wlin_ref, blin_ref, out_ref, pool_acc, gcnt_acc):
    i = pl.program_id(0)

    @pl.when(i == 0)
    def _():
        pool_acc[...] = jnp.zeros_like(pool_acc)
        gcnt_acc[...] = jnp.zeros_like(gcnt_acc)

    cnt = (cnt_ref[0, :, 0] + cnt_ref[1, :, 0]).astype(jnp.float32)
    acc = acc_ref[...].astype(jnp.float32)
    agg = (acc[0, 0] + acc[0, 1] + acc[1, 0]
           + acc[1, 1]) / jnp.clip(cnt, 1.0, None)[:, None]
    h2 = (jnp.dot(agg, wl_ref[...], preferred_element_type=jnp.float32)
          + bl_ref[...]
          + jnp.dot(h_ref[...], wr_ref[...], preferred_element_type=jnp.float32))
    b = batch_ref[0, 0, :]
    gids = lax.broadcasted_iota(jnp.int32, (N_GRAPHS, BN), 0)
    m = (gids == b[None, :]).astype(jnp.float32)
    pool_acc[...] += jnp.dot(m, h2, preferred_element_type=jnp.float32)
    gcnt_acc[...] += jnp.broadcast_to(jnp.sum(m, axis=1)[:, None], (N_GRAPHS, D))

    @pl.when(i == pl.num_programs(0) - 1)
    def _():
        pooled = pool_acc[...] / jnp.clip(gcnt_acc[...], 1.0, None)
        out_ref[...] = (jnp.dot(pooled, wlin_ref[...],
                                preferred_element_type=jnp.float32) + blin_ref[...])


def _make_tc_final():
    return pl.pallas_call(
        _tc_final_body,
        grid=(GRID,),
        in_specs=[
            pl.BlockSpec((NC, 2, BN, D), lambda i: (0, 0, i, 0)),
            pl.BlockSpec((NC, BN, 16), lambda i: (0, i, 0)),
            pl.BlockSpec((BN, D), lambda i: (i, 0)),
            pl.BlockSpec((D, D), lambda i: (0, 0)),
            pl.BlockSpec((D, D), lambda i: (0, 0)),
            pl.BlockSpec((1, D), lambda i: (0, 0)),
            pl.BlockSpec((1, 1, BN), lambda i: (i, 0, 0)),
            pl.BlockSpec((D, D), lambda i: (0, 0)),
            pl.BlockSpec((1, D), lambda i: (0, 0)),
        ],
        out_specs=pl.BlockSpec((N_GRAPHS, D), lambda i: (0, 0)),
        out_shape=jax.ShapeDtypeStruct((N_GRAPHS, D), jnp.float32),
        scratch_shapes=[
            pltpu.VMEM((N_GRAPHS, D), jnp.float32),
            pltpu.VMEM((N_GRAPHS, D), jnp.float32),
        ],
    )


_sc_agg_a = _make_sc_agg(D, True)
_sc_agg_b = _make_sc_agg(D, False)
_tc_layer1 = _make_tc_layer()
_tc_final = _make_tc_final()


def kernel(x, edge_index, batch, Wl1, bl1, Wr1, Wl2, bl2, Wr2, Wlin, blin):
    x = x.astype(jnp.float32)
    src = edge_index[0].astype(jnp.int32)
    dst = edge_index[1].astype(jnp.int32)
    src_p = jnp.concatenate(
        [src, jnp.zeros((E_PAD - N_EDGES,),
                        jnp.int32)]).reshape(N_CHUNKS, 1, CHUNK)
    pad_dst = N_NODES + jnp.arange(E_PAD - N_EDGES, dtype=jnp.int32) % (
        N_PAD - N_NODES)
    dst_p = jnp.concatenate([dst, pad_dst]).reshape(N_CHUNKS, 1, CHUNK)
    x_p = jnp.concatenate(
        [x, jnp.zeros((N_PAD - N_NODES, D), jnp.float32)], axis=0)
    x_bf = x_p.astype(jnp.bfloat16)
    zeros_b = jnp.zeros((N_PAD, D), jnp.bfloat16)
    zeros16 = jnp.zeros((N_PAD, 16), jnp.bfloat16)
    ones16 = jnp.ones((CHUNK, 16), jnp.bfloat16)
    batch_p = jnp.concatenate(
        [batch.astype(jnp.int32),
         jnp.full((N_PAD - N_NODES,), N_GRAPHS, jnp.int32)]).reshape(GRID, 1, BN)

    wl1t = Wl1.T.astype(jnp.float32)
    wr1t = Wr1.T.astype(jnp.float32)
    wl2t = Wl2.T.astype(jnp.float32)
    wr2t = Wr2.T.astype(jnp.float32)
    bl1r = bl1.astype(jnp.float32).reshape(1, D)
    bl2r = bl2.astype(jnp.float32).reshape(1, D)
    wlint = jnp.pad(Wlin.T.astype(jnp.float32), ((0, 0), (0, D - N_CLASSES)))
    blinr = jnp.pad(blin.astype(jnp.float32), (0, D - N_CLASSES)).reshape(1, D)

    acc1, cnt16 = _sc_agg_a(x_bf, src_p, dst_p, zeros_b, zeros16, ones16)
    h, hbf = _tc_layer1(acc1, cnt16, x_p, wl1t, wr1t, bl1r)
    (acc2,) = _sc_agg_b(hbf, src_p, dst_p, zeros_b)
    out = _tc_final(acc2, cnt16, h, wl2t, wr2t, bl2r, batch_p, wlint, blinr)
    return out[:, :N_CLASSES]
